# baseline (speedup 1.0000x reference)
import math

import jax
import jax.numpy as jnp
from jax.experimental import pallas as pl
from jax.experimental.pallas import tpu as pltpu

_B = 16
_N = 128
_D = 128
_NB = 8
_R_MAX = 5.0
_MACE_OUT = 640
_HID = 512
_CHUNK = 8
_NCHUNK = _N // _CHUNK
_PERSTEP = 2


def _silu(v):
    return 0.5 * v * (1.0 + jnp.tanh(0.5 * v))


def _pair_planes(pos):
    n = _N
    px = pos[:, 0:1]
    py = pos[:, 1:2]
    pz = pos[:, 2:3]
    dx = px - px.reshape(1, n)
    dy = py - py.reshape(1, n)
    dz = pz - pz.reshape(1, n)
    r2 = dx * dx + dy * dy + dz * dz
    ii = jax.lax.broadcasted_iota(jnp.int32, (n, n), 0)
    jj = jax.lax.broadcasted_iota(jnp.int32, (n, n), 1)
    eye = ii == jj
    r = jnp.sqrt(jnp.where(eye, 1.0, r2))
    x = r * (1.0 / _R_MAX)
    x5 = x * x * x * x * x
    cut = 1.0 - 21.0 * x5 + 35.0 * x5 * x - 15.0 * x5 * x * x
    cut = jnp.where(x < 1.0, cut, 0.0)
    cut = jnp.where(eye, 0.0, cut)
    coef = math.sqrt(2.0 / _R_MAX) * cut / r
    a = (math.pi / _R_MAX) * r
    s1 = jnp.sin(a)
    c2 = 2.0 * jnp.cos(a)
    planes = [coef * s1]
    prev, cur = s1, c2 * s1
    for _ in range(_NB - 2):
        planes.append(coef * cur)
        prev, cur = cur, c2 * cur - prev
    planes.append(coef * cur)
    return planes


def _fwd(pos_ref, emb_ref, w1big_ref, wr2_ref, wr3_ref, wr4_ref,
         wmsg0_ref, wupd0_ref, wmsg1_ref, wupd1_ref,
         wproj_ref, wmlp1_ref, b1_ref, wmlp2_ref, b2_ref, wmlp3_ref, b3_ref,
         out_ref, rw_sc):
    n = _N
    planes_all = [_pair_planes(pos_ref[s]) for s in range(_PERSTEP)]
    w1big = w1big_ref[...]
    wr2 = wr2_ref[...]
    wr3 = wr3_ref[...]
    wr4 = wr4_ref[...]
    rows0 = [[] for _ in range(_PERSTEP)]
    csums = [[] for _ in range(_PERSTEP)]
    for c in range(_NCHUNK):
        sl = slice(_CHUNK * c, _CHUNK * (c + 1))
        lo = _CHUNK * c
        for s in range(_PERSTEP):
            ef = jnp.concatenate(
                [p[lo:, sl] for p in planes_all[s]], axis=1)
            z1 = jnp.dot(ef, w1big,
                         precision=jax.lax.Precision.DEFAULT)
            cs = None
            for t in range(_CHUNK):
                i = _CHUNK * c + t
                zt = _silu(z1[:, 128 * t:128 * (t + 1)])
                zt = _silu(jnp.dot(zt, wr2,
                                   precision=jax.lax.Precision.DEFAULT))
                zt = _silu(jnp.dot(zt, wr3,
                                   precision=jax.lax.Precision.DEFAULT))
                rwt = jnp.dot(zt, wr4,
                              precision=jax.lax.Precision.DEFAULT)
                rw_sc[s, i, lo:, :] = rwt[:, _D:]
                if lo:
                    rw_sc[s, i, :lo, :] = jnp.zeros((lo, _D), jnp.float32)
                rw0 = rwt[:, :_D]
                rows0[s].append(jnp.sum(rw0, axis=0, keepdims=True))
                cs = rw0 if cs is None else cs + rw0
            csums[s].append(cs)
    emb = emb_ref[...]
    v0 = emb @ wmsg0_ref[...]
    u0 = emb @ wupd0_ref[...]
    for s in range(_PERSTEP):
        mir0 = csums[s][0]
        for c in range(1, _NCHUNK):
            lo = _CHUNK * c
            mir0 = mir0 + jnp.concatenate(
                [jnp.zeros((lo, _D), jnp.float32), csums[s][c]], axis=0)
        dc0 = jnp.concatenate([csums[s][c][:_CHUNK] for c in range(_NCHUNK)],
                              axis=0)
        ax1 = jnp.concatenate(rows0[s], axis=0)
        agg0 = (ax1 + mir0 - dc0) * v0
        h1 = u0 + agg0
        hm1 = h1 @ wmsg1_ref[...]
        hm1pl = hm1.reshape(n, 1, _D)
        rw1 = rw_sc[s]
        rw1w = rw1 * hm1pl
        mir1 = jnp.sum(rw1w, axis=0)
        dc1 = jnp.concatenate(
            [jnp.sum(rw1w[_CHUNK * b:_CHUNK * (b + 1),
                          _CHUNK * b:_CHUNK * (b + 1), :], axis=0)
             for b in range(_NCHUNK)], axis=0)
        agg1 = jnp.sum(rw1 * hm1[None, :, :], axis=1) + mir1 - dc1
        h2 = h1 @ wupd1_ref[...] + agg1
        nf = h1 @ wproj_ref[:_D, :] + h2 @ wproj_ref[_D:, :]
        o = jnp.maximum(nf @ wmlp1_ref[...] + b1_ref[...], 0.0)
        o = jnp.maximum(o @ wmlp2_ref[...] + b2_ref[...], 0.0)
        out_ref[s] = o @ wmlp3_ref[...] + b3_ref[...]


def _full(shape):
    nd = len(shape)
    return pl.BlockSpec(shape, lambda b: (0,) * nd)


def kernel(noisy_relative_positions, time, W_embed, Wr0_1, Wr0_2, Wr0_3,
           Wr0_4, Wmsg0, Wupd0, Wr1_1, Wr1_2, Wr1_3, Wr1_4, Wmsg1, Wupd1,
           Wproj, Wmlp1, bmlp1, Wmlp2, bmlp2, Wmlp3, bmlp3):
    del time
    pos = noisy_relative_positions
    z64 = jnp.zeros((64, 64), jnp.float32)
    z64_128 = jnp.zeros((64, _D), jnp.float32)
    Wr1c = jnp.concatenate([Wr0_1, Wr1_1], axis=1)
    Wr2c = jnp.block([[Wr0_2, z64], [z64, Wr1_2]])
    Wr3c = jnp.block([[Wr0_3, z64], [z64, Wr1_3]])
    Wr4c = jnp.block([[Wr0_4, z64_128], [z64_128, Wr1_4]])
    w1big = jnp.einsum('kc,ab->kabc', Wr1c, jnp.eye(_CHUNK, dtype=jnp.float32))
    w1big = w1big.reshape(_NB * _CHUNK, _CHUNK * _D)
    emb2 = W_embed[None, :]
    b1 = bmlp1[None, :]
    b2 = bmlp2[None, :]
    b3 = bmlp3[None, :]
    args = (pos, emb2, w1big, Wr2c, Wr3c, Wr4c, Wmsg0, Wupd0, Wmsg1, Wupd1,
            Wproj, Wmlp1, b1, Wmlp2, b2, Wmlp3, b3)
    in_specs = [pl.BlockSpec((_PERSTEP, _N, 3), lambda b: (b, 0, 0))]
    in_specs += [_full(a.shape) for a in args[1:]]
    return pl.pallas_call(
        _fwd,
        grid=(_B // _PERSTEP,),
        in_specs=in_specs,
        out_specs=pl.BlockSpec((_PERSTEP, _N, 3), lambda b: (b, 0, 0)),
        out_shape=jax.ShapeDtypeStruct((_B, _N, 3), jnp.float32),
        scratch_shapes=[pltpu.VMEM((_PERSTEP, _N, _N, _D), jnp.float32)],
        compiler_params=pltpu.CompilerParams(
            dimension_semantics=("parallel",)),
    )(*args)


# speedup vs baseline: 2.9105x; 2.9105x over previous
import math

import jax
import jax.numpy as jnp
from jax.experimental import pallas as pl
from jax.experimental.pallas import tpu as pltpu

_B = 16
_N = 128
_D = 128
_NB = 8
_R_MAX = 5.0
_MACE_OUT = 640
_HID = 512
_CHUNK = 16
_NCHUNK = _N // _CHUNK
_PERSTEP = 2


def _silu(v):
    return 0.5 * v * (1.0 + jnp.tanh(0.5 * v))


def _pair_planes(pos):
    n = _N
    px = pos[:, 0:1]
    py = pos[:, 1:2]
    pz = pos[:, 2:3]
    dx = px - px.reshape(1, n)
    dy = py - py.reshape(1, n)
    dz = pz - pz.reshape(1, n)
    r2 = dx * dx + dy * dy + dz * dz
    ii = jax.lax.broadcasted_iota(jnp.int32, (n, n), 0)
    jj = jax.lax.broadcasted_iota(jnp.int32, (n, n), 1)
    eye = ii == jj
    r = jnp.sqrt(jnp.where(eye, 1.0, r2))
    x = r * (1.0 / _R_MAX)
    x5 = x * x * x * x * x
    cut = 1.0 - 21.0 * x5 + 35.0 * x5 * x - 15.0 * x5 * x * x
    cut = jnp.where(x < 1.0, cut, 0.0)
    cut = jnp.where(eye, 0.0, cut)
    coef = math.sqrt(2.0 / _R_MAX) * cut / r
    a = (math.pi / _R_MAX) * r
    s1 = jnp.sin(a)
    c2 = 2.0 * jnp.cos(a)
    planes = [coef * s1]
    prev, cur = s1, c2 * s1
    for _ in range(_NB - 2):
        planes.append(coef * cur)
        prev, cur = cur, c2 * cur - prev
    planes.append(coef * cur)
    return planes


def _fwd(pos_ref, emb_ref, w1big_ref, wr2_ref, wr3_ref, wr4_ref,
         wmsg0_ref, wupd0_ref, wmsg1_ref, wupd1_ref,
         wproj_ref, wmlp1_ref, b1_ref, wmlp2_ref, b2_ref, wmlp3_ref, b3_ref,
         out_ref, rw_sc, stage_a, stage_b):
    n = _N
    planes_all = [_pair_planes(pos_ref[s]) for s in range(_PERSTEP)]
    w1big = w1big_ref[...]
    wr2 = wr2_ref[...]
    wr3 = wr3_ref[...]
    wr4 = wr4_ref[...]
    rows0 = [[] for _ in range(_PERSTEP)]
    csums = [[] for _ in range(_PERSTEP)]
    for c in range(_NCHUNK):
        sl = slice(_CHUNK * c, _CHUNK * (c + 1))
        lo = _CHUNK * c
        m = _N - lo
        for s in range(_PERSTEP):
            ef = jnp.concatenate(
                [p[lo:, sl] for p in planes_all[s]], axis=1)
            z1 = jnp.dot(ef, w1big,
                         precision=jax.lax.Precision.DEFAULT)
            stage_a[s, :m, :] = _silu(z1)
            for t in range(_CHUNK):
                tsl = slice(128 * t, 128 * (t + 1))
                stage_b[s, :m, tsl] = _silu(
                    jnp.dot(stage_a[s, :m, tsl], wr2,
                            precision=jax.lax.Precision.DEFAULT))
            for t in range(_CHUNK):
                tsl = slice(128 * t, 128 * (t + 1))
                stage_a[s, :m, tsl] = _silu(
                    jnp.dot(stage_b[s, :m, tsl], wr3,
                            precision=jax.lax.Precision.DEFAULT))
            cs = None
            for t in range(_CHUNK):
                i = _CHUNK * c + t
                tsl = slice(128 * t, 128 * (t + 1))
                rwt = jnp.dot(stage_a[s, :m, tsl], wr4,
                              precision=jax.lax.Precision.DEFAULT)
                rw_sc[s, i, lo:, :] = rwt[:, _D:]
                if lo:
                    rw_sc[s, i, :lo, :] = jnp.zeros((lo, _D), jnp.float32)
                rw0 = rwt[:, :_D]
                rows0[s].append(jnp.sum(rw0, axis=0, keepdims=True))
                cs = rw0 if cs is None else cs + rw0
            csums[s].append(cs)
    emb = emb_ref[...]
    v0 = emb @ wmsg0_ref[...]
    u0 = emb @ wupd0_ref[...]
    for s in range(_PERSTEP):
        mir0 = csums[s][0]
        for c in range(1, _NCHUNK):
            lo = _CHUNK * c
            mir0 = mir0 + jnp.concatenate(
                [jnp.zeros((lo, _D), jnp.float32), csums[s][c]], axis=0)
        dc0 = jnp.concatenate([csums[s][c][:_CHUNK] for c in range(_NCHUNK)],
                              axis=0)
        ax1 = jnp.concatenate(rows0[s], axis=0)
        agg0 = (ax1 + mir0 - dc0) * v0
        h1 = u0 + agg0
        hm1 = h1 @ wmsg1_ref[...]
        hm1pl = hm1.reshape(n, 1, _D)
        rw1 = rw_sc[s]
        rw1w = rw1 * hm1pl
        mir1 = jnp.sum(rw1w, axis=0)
        dc1 = jnp.concatenate(
            [jnp.sum(rw1w[_CHUNK * b:_CHUNK * (b + 1),
                          _CHUNK * b:_CHUNK * (b + 1), :], axis=0)
             for b in range(_NCHUNK)], axis=0)
        agg1 = jnp.sum(rw1 * hm1[None, :, :], axis=1) + mir1 - dc1
        h2 = h1 @ wupd1_ref[...] + agg1
        nf = h1 @ wproj_ref[:_D, :] + h2 @ wproj_ref[_D:, :]
        o = jnp.maximum(nf @ wmlp1_ref[...] + b1_ref[...], 0.0)
        o = jnp.maximum(o @ wmlp2_ref[...] + b2_ref[...], 0.0)
        out_ref[s] = o @ wmlp3_ref[...] + b3_ref[...]


def _full(shape):
    nd = len(shape)
    return pl.BlockSpec(shape, lambda b: (0,) * nd)


def kernel(noisy_relative_positions, time, W_embed, Wr0_1, Wr0_2, Wr0_3,
           Wr0_4, Wmsg0, Wupd0, Wr1_1, Wr1_2, Wr1_3, Wr1_4, Wmsg1, Wupd1,
           Wproj, Wmlp1, bmlp1, Wmlp2, bmlp2, Wmlp3, bmlp3):
    del time
    pos = noisy_relative_positions
    z64 = jnp.zeros((64, 64), jnp.float32)
    z64_128 = jnp.zeros((64, _D), jnp.float32)
    Wr1c = jnp.concatenate([Wr0_1, Wr1_1], axis=1)
    Wr2c = jnp.block([[Wr0_2, z64], [z64, Wr1_2]])
    Wr3c = jnp.block([[Wr0_3, z64], [z64, Wr1_3]])
    Wr4c = jnp.block([[Wr0_4, z64_128], [z64_128, Wr1_4]])
    w1big = jnp.einsum('kc,ab->kabc', Wr1c, jnp.eye(_CHUNK, dtype=jnp.float32))
    w1big = w1big.reshape(_NB * _CHUNK, _CHUNK * _D)
    emb2 = W_embed[None, :]
    b1 = bmlp1[None, :]
    b2 = bmlp2[None, :]
    b3 = bmlp3[None, :]
    args = (pos, emb2, w1big, Wr2c, Wr3c, Wr4c, Wmsg0, Wupd0, Wmsg1, Wupd1,
            Wproj, Wmlp1, b1, Wmlp2, b2, Wmlp3, b3)
    in_specs = [pl.BlockSpec((_PERSTEP, _N, 3), lambda b: (b, 0, 0))]
    in_specs += [_full(a.shape) for a in args[1:]]
    return pl.pallas_call(
        _fwd,
        grid=(_B // _PERSTEP,),
        in_specs=in_specs,
        out_specs=pl.BlockSpec((_PERSTEP, _N, 3), lambda b: (b, 0, 0)),
        out_shape=jax.ShapeDtypeStruct((_B, _N, 3), jnp.float32),
        scratch_shapes=[pltpu.VMEM((_PERSTEP, _N, _N, _D), jnp.float32),
                        pltpu.VMEM((_PERSTEP, _N, _CHUNK * _D), jnp.float32),
                        pltpu.VMEM((_PERSTEP, _N, _CHUNK * _D), jnp.float32)],
        compiler_params=pltpu.CompilerParams(
            dimension_semantics=("parallel",)),
    )(*args)
